# Initial kernel scaffold; baseline (speedup 1.0000x reference)
import jax
import jax.numpy as jnp
from jax.experimental import pallas as pl


def kernel(x, edge_index, W, gamma, beta):
    raise NotImplementedError("write your pallas kernel here")


# speedup vs baseline: 1.5163x; 1.5163x over previous
import functools

import jax
import jax.numpy as jnp
from jax import lax
from jax.experimental import pallas as pl
from jax.experimental.pallas import tpu as pltpu
from jax.experimental.pallas import tpu_sc as plsc

C = 128
N = 10000
K = 16
OUT = 128

NW = 32
PW = 320
NP = NW * PW
CH = 8
CE = CH * K
NCH = PW // CH


def _mm_body(x_ref, w_ref, at_ref, bt_ref):
    X = x_ref[...]
    W = w_ref[...]
    W1 = W[:, :C]
    W2 = W[:, C:]
    at_ref[...] = lax.dot_general(
        X, W1 + W2, (((0,), (1,)), ((), ())),
        preferred_element_type=jnp.float32)
    bt_ref[...] = lax.dot_general(
        X, W2, (((0,), (1,)), ((), ())),
        preferred_element_type=jnp.float32)


def _matmuls(x2d, W):
    return pl.pallas_call(
        _mm_body,
        out_shape=[
            jax.ShapeDtypeStruct((N, OUT), jnp.float32),
            jax.ShapeDtypeStruct((N, OUT), jnp.float32),
        ],
    )(x2d, W)


def _sc_body(table_hbm, idx_hbm, min_hbm, s1_hbm, s2_hbm,
             idx_v, rows_v, mn_v, s1_v, s2_v, sem):
    wid = lax.axis_index("s") * 2 + lax.axis_index("c")
    pltpu.sync_copy(idx_hbm.at[wid], idx_v)

    def chunk(c, carry):
        pltpu.async_copy(table_hbm.at[idx_v.at[c]], rows_v, sem).wait()

        def node(i, carry2):
            r0 = i * K
            for v in range(OUT // 16):
                sl = pl.ds(v * 16, 16)
                m = rows_v[r0, sl]
                s = m
                q = m * m
                for r in range(1, K):
                    t = rows_v[r0 + r, sl]
                    m = jnp.minimum(m, t)
                    s = s + t
                    q = q + t * t
                mn_v[i, sl] = m
                s1_v[i, sl] = s
                s2_v[i, sl] = q
            return carry2

        lax.fori_loop(0, CH, node, 0)
        row = wid * PW + c * CH
        pltpu.sync_copy(mn_v, min_hbm.at[pl.ds(row, CH)])
        pltpu.sync_copy(s1_v, s1_hbm.at[pl.ds(row, CH)])
        pltpu.sync_copy(s2_v, s2_hbm.at[pl.ds(row, CH)])
        return carry

    lax.fori_loop(0, NCH, chunk, 0)


def _sc_gather(bt, idx3):
    mesh = plsc.VectorSubcoreMesh(core_axis_name="c", subcore_axis_name="s")
    f = functools.partial(
        pl.kernel,
        out_type=[
            jax.ShapeDtypeStruct((NP, OUT), jnp.float32),
            jax.ShapeDtypeStruct((NP, OUT), jnp.float32),
            jax.ShapeDtypeStruct((NP, OUT), jnp.float32),
        ],
        mesh=mesh,
        scratch_types=[
            pltpu.VMEM((NCH, CE), jnp.int32),
            pltpu.VMEM((CE, OUT), jnp.float32),
            pltpu.VMEM((CH, OUT), jnp.float32),
            pltpu.VMEM((CH, OUT), jnp.float32),
            pltpu.VMEM((CH, OUT), jnp.float32),
            pltpu.SemaphoreType.DMA,
        ],
    )(_sc_body)
    return f(bt, idx3)


def _fin_body(at_ref, mn_ref, s1_ref, s2_ref, g_ref, b_ref, o_ref):
    At = at_ref[...]
    S1 = s1_ref[...]
    inv_nk = 1.0 / (N * K)
    sumA = jnp.sum(At, axis=0, keepdims=True)
    sumA2 = jnp.sum(At * At, axis=0, keepdims=True)
    sAS1 = jnp.sum(At * S1, axis=0, keepdims=True)
    sS1 = jnp.sum(S1, axis=0, keepdims=True)
    sS2 = jnp.sum(s2_ref[...], axis=0, keepdims=True)
    mean = (K * sumA - sS1) * inv_nk
    e2 = (K * sumA2 - 2.0 * sAS1 + sS2) * inv_nk
    var = e2 - mean * mean
    inv = lax.rsqrt(var + 1e-5)
    h = (At - mn_ref[...] - mean) * (inv * g_ref[...]) + b_ref[...]
    o_ref[...] = jnp.where(h >= 0, h, 0.2 * h)


def _finalize(at, mn, s1, s2, gamma, beta):
    return pl.pallas_call(
        _fin_body,
        out_shape=jax.ShapeDtypeStruct((N, OUT), jnp.float32),
    )(at, mn, s1, s2, gamma.reshape(1, OUT), beta.reshape(1, OUT))


def kernel(x, edge_index, W, gamma, beta):
    x2d = x.reshape(C, N)
    at, bt = _matmuls(x2d, W)
    idx_flat = edge_index.reshape(N * K)
    idx_pad = jnp.pad(idx_flat, (0, NP * K - N * K))
    idx3 = idx_pad.reshape(NW, NCH, CE)
    mn, s1, s2 = _sc_gather(bt, idx3)
    res = _finalize(at, mn[:N], s1[:N], s2[:N], gamma, beta)
    return jnp.transpose(res).reshape(1, OUT, N, 1)


# speedup vs baseline: 1.8684x; 1.2323x over previous
import functools

import jax
import jax.numpy as jnp
from jax import lax
from jax.experimental import pallas as pl
from jax.experimental.pallas import tpu as pltpu
from jax.experimental.pallas import tpu_sc as plsc

C = 128
N = 10000
K = 16
OUT = 128

NW = 32
PW = 320
NP = NW * PW
CH = 8
CE = CH * K
NCH = PW // CH
NV = OUT // 16


def _mm_body(x_ref, w_ref, at_ref, bt_ref, sa_ref, sa2_ref):
    X = x_ref[...]
    W = w_ref[...]
    W1 = W[:, :C]
    W2 = W[:, C:]
    At = lax.dot_general(X, W1 + W2, (((0,), (1,)), ((), ())),
                         preferred_element_type=jnp.float32)
    Bt = lax.dot_general(X, W2, (((0,), (1,)), ((), ())),
                         preferred_element_type=jnp.float32)
    at_ref[pl.ds(0, N), :] = At
    at_ref[pl.ds(N, NP - N), :] = jnp.zeros((NP - N, OUT), jnp.float32)
    bt_ref[pl.ds(0, N), :] = Bt
    bt_ref[pl.ds(N, NP - N), :] = jnp.zeros((NP - N, OUT), jnp.float32)
    sa_ref[...] = jnp.sum(At, axis=0, keepdims=True)
    sa2_ref[...] = jnp.sum(At * At, axis=0, keepdims=True)


def _matmuls(x2d, W):
    return pl.pallas_call(
        _mm_body,
        out_shape=[
            jax.ShapeDtypeStruct((NP, OUT), jnp.float32),
            jax.ShapeDtypeStruct((NP, OUT), jnp.float32),
            jax.ShapeDtypeStruct((1, OUT), jnp.float32),
            jax.ShapeDtypeStruct((1, OUT), jnp.float32),
        ],
    )(x2d, W)


def _sc_body(table_hbm, ap_hbm, idx_hbm, pre_hbm, s1_hbm, s2_hbm, as_hbm,
             idx_v, rows0, rows1, a0, a1, pre_v, p1_v, p2_v, p3_v,
             gsem0, gsem1, asem0, asem1):
    wid = lax.axis_index("s") * 2 + lax.axis_index("c")
    nbase = wid * PW
    pltpu.sync_copy(idx_hbm.at[wid], idx_v)

    rows = (rows0, rows1)
    abufs = (a0, a1)
    gsems = (gsem0, gsem1)
    asems = (asem0, asem1)

    def gcopy(c, b):
        return pltpu.make_async_copy(
            table_hbm.at[idx_v.at[c]], rows[b], gsems[b])

    def acopy(c, b):
        return pltpu.make_async_copy(
            ap_hbm.at[pl.ds(nbase + c * CH, CH)], abufs[b], asems[b])

    gcopy(0, 0).start()
    acopy(0, 0).start()
    gcopy(1, 1).start()
    acopy(1, 1).start()

    zero = jnp.zeros((16,), jnp.float32)
    carry0 = (zero,) * (3 * NV)

    def chunk2(c2, carry):
        for b in range(2):
            c = c2 * 2 + b
            gcopy(c, b).wait()
            acopy(c, b).wait()
            rv = rows[b]
            av = abufs[b]

            def node(i, cr):
                r0 = i * K
                out = []
                for v in range(NV):
                    sl = pl.ds(v * 16, 16)
                    m = rv[r0, sl]
                    s = m
                    q = m * m
                    for r in range(1, K):
                        t = rv[r0 + r, sl]
                        m = jnp.minimum(m, t)
                        s = s + t
                        q = q + t * t
                    a = av[i, sl]
                    pre_v[c * CH + i, sl] = a - m
                    out.append(cr[v] + s)
                    out.append(cr[NV + v] + q)
                    out.append(cr[2 * NV + v] + a * s)
                return tuple(out[0::3]) + tuple(out[1::3]) + tuple(out[2::3])

            carry = lax.fori_loop(0, CH, node, carry)

            @pl.when(c2 < NCH // 2 - 1)
            def _():
                gcopy(c + 2, b).start()
                acopy(c + 2, b).start()
        return carry

    carry = lax.fori_loop(0, NCH // 2, chunk2, carry0)

    for v in range(NV):
        sl = pl.ds(v * 16, 16)
        p1_v[0, sl] = carry[v]
        p2_v[0, sl] = carry[NV + v]
        p3_v[0, sl] = carry[2 * NV + v]
    pltpu.sync_copy(pre_v, pre_hbm.at[pl.ds(nbase, PW)])
    pltpu.sync_copy(p1_v, s1_hbm.at[pl.ds(wid, 1)])
    pltpu.sync_copy(p2_v, s2_hbm.at[pl.ds(wid, 1)])
    pltpu.sync_copy(p3_v, as_hbm.at[pl.ds(wid, 1)])


def _sc_gather(bt, at, idx3):
    mesh = plsc.VectorSubcoreMesh(core_axis_name="c", subcore_axis_name="s")
    f = functools.partial(
        pl.kernel,
        out_type=[
            jax.ShapeDtypeStruct((NP, OUT), jnp.float32),
            jax.ShapeDtypeStruct((NW, OUT), jnp.float32),
            jax.ShapeDtypeStruct((NW, OUT), jnp.float32),
            jax.ShapeDtypeStruct((NW, OUT), jnp.float32),
        ],
        mesh=mesh,
        scratch_types=[
            pltpu.VMEM((NCH, CE), jnp.int32),
            pltpu.VMEM((CE, OUT), jnp.float32),
            pltpu.VMEM((CE, OUT), jnp.float32),
            pltpu.VMEM((CH, OUT), jnp.float32),
            pltpu.VMEM((CH, OUT), jnp.float32),
            pltpu.VMEM((PW, OUT), jnp.float32),
            pltpu.VMEM((1, OUT), jnp.float32),
            pltpu.VMEM((1, OUT), jnp.float32),
            pltpu.VMEM((1, OUT), jnp.float32),
            pltpu.SemaphoreType.DMA,
            pltpu.SemaphoreType.DMA,
            pltpu.SemaphoreType.DMA,
            pltpu.SemaphoreType.DMA,
        ],
    )(_sc_body)
    return f(bt, at, idx3)


def _fin_body(pre_ref, s1_ref, s2_ref, as_ref, sa_ref, sa2_ref,
              g_ref, b_ref, o_ref):
    inv_nk = 1.0 / (N * K)
    sS1 = jnp.sum(s1_ref[...], axis=0, keepdims=True)
    sS2 = jnp.sum(s2_ref[...], axis=0, keepdims=True)
    sAS1 = jnp.sum(as_ref[...], axis=0, keepdims=True)
    mean = (K * sa_ref[...] - sS1) * inv_nk
    e2 = (K * sa2_ref[...] - 2.0 * sAS1 + sS2) * inv_nk
    var = e2 - mean * mean
    inv = lax.rsqrt(var + 1e-5)
    pre = pre_ref[pl.ds(0, N), :]
    h = (pre - mean) * (inv * g_ref[...]) + b_ref[...]
    o_ref[...] = jnp.where(h >= 0, h, 0.2 * h)


def _finalize(pre, s1p, s2p, asp, sa, sa2, gamma, beta):
    return pl.pallas_call(
        _fin_body,
        out_shape=jax.ShapeDtypeStruct((N, OUT), jnp.float32),
    )(pre, s1p, s2p, asp, sa, sa2,
      gamma.reshape(1, OUT), beta.reshape(1, OUT))


def kernel(x, edge_index, W, gamma, beta):
    x2d = x.reshape(C, N)
    at, bt, sa, sa2 = _matmuls(x2d, W)
    idx_flat = edge_index.reshape(N * K)
    idx_pad = jnp.pad(idx_flat, (0, NP * K - N * K), constant_values=N)
    idx3 = idx_pad.reshape(NW, NCH, CE)
    pre, s1p, s2p, asp = _sc_gather(bt, at, idx3)
    res = _finalize(pre, s1p, s2p, asp, sa, sa2, gamma, beta)
    return jnp.transpose(res).reshape(1, OUT, N, 1)


# speedup vs baseline: 5.3976x; 2.8889x over previous
import functools

import jax
import jax.numpy as jnp
from jax import lax
from jax.experimental import pallas as pl
from jax.experimental.pallas import tpu as pltpu
from jax.experimental.pallas import tpu_sc as plsc

C = 128
N = 10000
K = 16
OUT = 128

NW = 32
PW = 320
NP = NW * PW
CH = 8
CE = CH * K
NCH = PW // CH
NV = OUT // 16


def _mm_body(x_ref, w_ref, at_ref, bt_ref, sa_ref, sa2_ref):
    X = x_ref[...]
    W = w_ref[...]
    W1 = W[:, :C]
    W2 = W[:, C:]
    At = lax.dot_general(X, W1 + W2, (((0,), (1,)), ((), ())),
                         preferred_element_type=jnp.float32)
    Bt = lax.dot_general(X, W2, (((0,), (1,)), ((), ())),
                         preferred_element_type=jnp.float32)
    at_ref[pl.ds(0, N), :] = At
    at_ref[pl.ds(N, NP - N), :] = jnp.zeros((NP - N, OUT), jnp.float32)
    bt_ref[pl.ds(0, N), :] = Bt
    bt_ref[pl.ds(N, NP - N), :] = jnp.zeros((NP - N, OUT), jnp.float32)
    sa_ref[...] = jnp.sum(At, axis=0, keepdims=True)
    sa2_ref[...] = jnp.sum(At * At, axis=0, keepdims=True)


def _matmuls(x2d, W):
    return pl.pallas_call(
        _mm_body,
        out_shape=[
            jax.ShapeDtypeStruct((NP, OUT), jnp.float32),
            jax.ShapeDtypeStruct((NP, OUT), jnp.float32),
            jax.ShapeDtypeStruct((1, OUT), jnp.float32),
            jax.ShapeDtypeStruct((1, OUT), jnp.float32),
        ],
    )(x2d, W)


def _sc_body(table_hbm, ap_hbm, idx_hbm, pre_hbm, s1_hbm, s2_hbm, as_hbm,
             idx_v, rows0, rows1, a0, a1, pre_v, p1_v, p2_v, p3_v,
             gsem0, gsem1, asem0, asem1):
    sid = lax.axis_index("s")
    wid = sid * 2 + lax.axis_index("c")
    nbase = wid * PW
    pltpu.sync_copy(idx_hbm.at[wid], idx_v)

    rows = (rows0, rows1)
    abufs = (a0, a1)
    gsems = (gsem0, gsem1)
    asems = (asem0, asem1)

    def gcopy(c, b):
        return pltpu.make_async_copy(
            table_hbm.at[idx_v.at[c]], rows[b], gsems[b])

    def acopy(c, b):
        return pltpu.make_async_copy(
            ap_hbm.at[pl.ds(nbase + c * CH, CH)], abufs[b], asems[b])

    gcopy(0, 0).start()
    acopy(0, 0).start()
    gcopy(1, 1).start()
    acopy(1, 1).start()

    zero = jnp.zeros((16,), jnp.float32)
    carry0 = (zero,) * (3 * NV)

    def chunk2(c2, carry):
        for b in range(2):
            c = c2 * 2 + b
            gcopy(c, b).wait()
            acopy(c, b).wait()
            rv = rows[b]
            av = abufs[b]

            def node(i, cr):
                r0 = i * K
                out = []
                for v in range(NV):
                    sl = pl.ds(v * 16, 16)
                    m = rv[r0, sl]
                    s = m
                    q = m * m
                    for r in range(1, K):
                        t = rv[r0 + r, sl]
                        m = jnp.minimum(m, t)
                        s = s + t
                        q = q + t * t
                    a = av[i, sl]
                    pre_v[c * CH + i, sl] = a - m
                    out.append(cr[v] + s)
                    out.append(cr[NV + v] + q)
                    out.append(cr[2 * NV + v] + a * s)
                return tuple(out[0::3]) + tuple(out[1::3]) + tuple(out[2::3])

            carry = lax.fori_loop(0, CH, node, carry)

            @pl.when(c2 < NCH // 2 - 1)
            def _():
                gcopy(c + 2, b).start()
                acopy(c + 2, b).start()
        return carry

    carry = lax.fori_loop(0, NCH // 2, chunk2, carry0)

    for v in range(NV):
        sl = pl.ds(v * 16, 16)
        p1_v[0, sl] = carry[v]
        p2_v[0, sl] = carry[NV + v]
        p3_v[0, sl] = carry[2 * NV + v]
    pltpu.sync_copy(pre_v, pre_hbm.at[pl.ds(nbase, PW)])
    pltpu.sync_copy(p1_v, s1_hbm.at[pl.ds(wid, 1)])
    pltpu.sync_copy(p2_v, s2_hbm.at[pl.ds(wid, 1)])
    pltpu.sync_copy(p3_v, as_hbm.at[pl.ds(wid, 1)])


def _sc_gather(bt, at, idx3):
    mesh = plsc.VectorSubcoreMesh(core_axis_name="c", subcore_axis_name="s")
    f = functools.partial(
        pl.kernel,
        out_type=[
            jax.ShapeDtypeStruct((NP, OUT), jnp.float32),
            jax.ShapeDtypeStruct((NW, OUT), jnp.float32),
            jax.ShapeDtypeStruct((NW, OUT), jnp.float32),
            jax.ShapeDtypeStruct((NW, OUT), jnp.float32),
        ],
        mesh=mesh,
        scratch_types=[
            pltpu.VMEM((NCH, CE), jnp.int32),
            pltpu.VMEM((CE, OUT), jnp.float32),
            pltpu.VMEM((CE, OUT), jnp.float32),
            pltpu.VMEM((CH, OUT), jnp.float32),
            pltpu.VMEM((CH, OUT), jnp.float32),
            pltpu.VMEM((PW, OUT), jnp.float32),
            pltpu.VMEM((1, OUT), jnp.float32),
            pltpu.VMEM((1, OUT), jnp.float32),
            pltpu.VMEM((1, OUT), jnp.float32),
            pltpu.SemaphoreType.DMA,
            pltpu.SemaphoreType.DMA,
            pltpu.SemaphoreType.DMA,
            pltpu.SemaphoreType.DMA,
        ],
    )(_sc_body)
    return f(bt, at, idx3)


def _fin_body(pre_ref, s1_ref, s2_ref, as_ref, sa_ref, sa2_ref,
              g_ref, b_ref, o_ref):
    inv_nk = 1.0 / (N * K)
    sS1 = jnp.sum(s1_ref[...], axis=0, keepdims=True)
    sS2 = jnp.sum(s2_ref[...], axis=0, keepdims=True)
    sAS1 = jnp.sum(as_ref[...], axis=0, keepdims=True)
    mean = (K * sa_ref[...] - sS1) * inv_nk
    e2 = (K * sa2_ref[...] - 2.0 * sAS1 + sS2) * inv_nk
    var = e2 - mean * mean
    inv = lax.rsqrt(var + 1e-5)
    pre = pre_ref[pl.ds(0, N), :]
    h = (pre - mean) * (inv * g_ref[...]) + b_ref[...]
    o_ref[...] = jnp.where(h >= 0, h, 0.2 * h)


def _finalize(pre, s1p, s2p, asp, sa, sa2, gamma, beta):
    return pl.pallas_call(
        _fin_body,
        out_shape=jax.ShapeDtypeStruct((N, OUT), jnp.float32),
    )(pre, s1p, s2p, asp, sa, sa2,
      gamma.reshape(1, OUT), beta.reshape(1, OUT))


def kernel(x, edge_index, W, gamma, beta):
    x2d = x.reshape(C, N)
    at, bt, sa, sa2 = _matmuls(x2d, W)
    idx_flat = edge_index.reshape(N * K)
    npad = NP * K - N * K
    pad_vals = N + (jnp.arange(npad, dtype=jnp.int32) % (NP - N))
    idx_pad = jnp.concatenate([idx_flat, pad_vals])
    idx3 = idx_pad.reshape(NW, NCH, CE)
    pre, s1p, s2p, asp = _sc_gather(bt, at, idx3)
    res = _finalize(pre, s1p, s2p, asp, sa, sa2, gamma, beta)
    return jnp.transpose(res).reshape(1, OUT, N, 1)
